# Initial kernel scaffold; baseline (speedup 1.0000x reference)
import jax
import jax.numpy as jnp
from jax.experimental import pallas as pl


def kernel(x, edge_index, batch, enc_W1, enc_b1, enc_W2, enc_b2, r_W1, r_b1, r_ln_g, r_ln_b, r_W2, r_b2, size_centers, exp_rel_W1, exp_rel_b1, exp_root_W1, exp_rel_W2, exp_rel_b2, exp_root_W2):
    raise NotImplementedError("write your pallas kernel here")


# speedup vs baseline: 10.0487x; 10.0487x over previous
import functools

import jax
import jax.numpy as jnp
from jax import lax
from jax.experimental import pallas as pl
from jax.experimental.pallas import tpu as pltpu
from jax.experimental.pallas import tpu_sc as plsc

N = 10000
E = 320000
D = 128
NEXP = 8
NG = 16
PAD = 240
NC, NS = 2, 16
CH = 80
NROWS = E // CH
ROWS_A = NROWS // (NC * NS)
ROWS_BC = NROWS // NS
HROWS = ROWS_BC // 2
NB = 10
BN = N // NB
NACC = 10240
RPT = NACC // NS
QF = RPT // CH


def _enc_body(x_ref, w1_ref, b1_ref, w2_ref, b2_ref, h_ref):
    t = jnp.dot(x_ref[...], w1_ref[...], preferred_element_type=jnp.float32)
    t = jnp.maximum(t + b1_ref[...], 0.0)
    h_ref[...] = jnp.dot(t, w2_ref[...], preferred_element_type=jnp.float32) + b2_ref[...]


def _encoder_call(x, w1, b1, w2, b2, interpret=False):
    return pl.pallas_call(
        _enc_body,
        grid=(NB,),
        in_specs=[
            pl.BlockSpec((BN, D), lambda i: (i, 0)),
            pl.BlockSpec((D, D), lambda i: (0, 0)),
            pl.BlockSpec((1, D), lambda i: (0, 0)),
            pl.BlockSpec((D, D), lambda i: (0, 0)),
            pl.BlockSpec((1, D), lambda i: (0, 0)),
        ],
        out_specs=pl.BlockSpec((BN, D), lambda i: (i, 0)),
        out_shape=jax.ShapeDtypeStruct((N, D), jnp.float32),
        interpret=interpret,
    )(x, w1, b1, w2, b2)


def _stats_body(bp_ref, src_ref, stats_ref):
    bp = bp_ref[...]
    src = src_ref[...]
    ns = [jnp.sum((bp == g).astype(jnp.float32)) for g in range(NG)]
    starts = [jnp.sum((bp < g).astype(jnp.int32)) - PAD for g in range(NG + 1)]
    es = []
    for g in range(NG):
        lo, hi = starts[g], starts[g + 1]
        es.append(jnp.sum(((src >= lo) & (src < hi)).astype(jnp.float32)))
    n_col = jnp.concatenate([v.reshape(1, 1) for v in ns], axis=0)
    e_col = jnp.concatenate([v.reshape(1, 1) for v in es], axis=0)
    gmax = jnp.max(bp)
    iota_g = lax.broadcasted_iota(jnp.int32, (NG, 1), 0)
    valid = iota_g <= gmax
    gcount = (gmax + 1).astype(jnp.float32)
    n_c = jnp.clip(n_col, 1.0, None)
    density = e_col / jnp.clip(n_c * (n_c - 1.0), 1.0, None)
    logn = jnp.log(n_c)
    logn_min = jnp.min(jnp.where(valid, logn, jnp.inf))
    logn_max = jnp.max(jnp.where(valid, logn, -jnp.inf))
    logn_norm = (logn - logn_min) / (logn_max - logn_min + 1e-6)
    feats = jnp.concatenate([n_c, e_col, density], axis=1)
    fmean = jnp.sum(jnp.where(valid, feats, 0.0), axis=0, keepdims=True) / gcount
    fvar = jnp.sum(jnp.where(valid, (feats - fmean) ** 2, 0.0), axis=0, keepdims=True) / gcount
    feats = (feats - fmean) / (jnp.sqrt(fvar) + 1e-6)
    stats_ref[...] = jnp.concatenate(
        [feats, logn_norm, jnp.zeros((NG, D - 4), jnp.float32)], axis=1)


def _stats_call(bp2, src2, interpret=False):
    return pl.pallas_call(
        _stats_body,
        out_shape=jax.ShapeDtypeStruct((NG, D), jnp.float32),
        interpret=interpret,
    )(bp2, src2)


def _moe_body(h_ref, a0_ref, a1_ref, bc_ref, st_ref, w1p_ref, rb1_ref,
              lng_ref, lnb_ref, w2_ref, rb2_ref, cent_ref,
              wr1_ref, br1_ref, wt1_ref, wr2_ref, br2_ref, wt2_ref,
              g_out, yp_out, isel_out, wsel_out):
    h = h_ref[...]
    agg1 = a0_ref[...] + a1_ref[...]
    bcol = bc_ref[...]
    iota_g = lax.broadcasted_iota(jnp.int32, (1, NG), 1)
    P = (bcol == iota_g).astype(jnp.float32)
    st = st_ref[...]
    feats3 = st[:, 0:3]
    logn16 = st[:, 3:4]
    w1p = w1p_ref[...]
    sf_w = jnp.dot(feats3, w1p[128:131, :], preferred_element_type=jnp.float32)
    t = (jnp.dot(h, w1p[0:128, :], preferred_element_type=jnp.float32)
         + jnp.dot(P, sf_w, preferred_element_type=jnp.float32,
                   precision=lax.Precision.HIGHEST) + rb1_ref[...])
    mu = jnp.mean(t, axis=1, keepdims=True)
    var = jnp.mean((t - mu) ** 2, axis=1, keepdims=True)
    tn = (t - mu) / jnp.sqrt(var + 1e-5) * lng_ref[...] + lnb_ref[...]
    learned = jnp.dot(jnp.maximum(tn, 0.0), w2_ref[...],
                      preferred_element_type=jnp.float32) + rb2_ref[...]
    logn_n = jnp.dot(P, logn16, preferred_element_type=jnp.float32,
                     precision=lax.Precision.HIGHEST)
    prior = -(logn_n - cent_ref[...]) ** 2
    logits = 0.7 * learned + 0.3 * prior
    z = logits - jnp.max(logits, axis=1, keepdims=True)
    pz = jnp.exp(z)
    probs = pz / jnp.sum(pz, axis=1, keepdims=True)
    io8 = lax.broadcasted_iota(jnp.int32, (1, NEXP), 1).astype(jnp.float32)
    m1 = jnp.max(probs, axis=1, keepdims=True)
    i1 = jnp.min(jnp.where(probs == m1, io8, 99.0), axis=1, keepdims=True)
    p2 = jnp.where(io8 == i1, -1.0, probs)
    m2 = jnp.max(p2, axis=1, keepdims=True)
    i2 = jnp.min(jnp.where(p2 == m2, io8, 99.0), axis=1, keepdims=True)
    s = m1 + m2 + 1e-8
    w1v = m1 / s
    w2v = m2 / s
    yp = jnp.zeros((BN, D), jnp.float32)
    for i in range(NEXP):
        h1 = jnp.maximum(
            jnp.dot(agg1, wr1_ref[i], preferred_element_type=jnp.float32)
            + br1_ref[i:i + 1, :]
            + jnp.dot(h, wt1_ref[i], preferred_element_type=jnp.float32), 0.0)
        g_out[i] = jnp.dot(h1, wr2_ref[i], preferred_element_type=jnp.float32)
        r2 = jnp.dot(h1, wt2_ref[i], preferred_element_type=jnp.float32)
        coef = (w1v * (i1 == float(i)).astype(jnp.float32)
                + w2v * (i2 == float(i)).astype(jnp.float32))
        yp = yp + coef * (r2 + br2_ref[i:i + 1, :])
    yp_out[...] = yp
    zcol = jnp.zeros((BN, NEXP - 2), jnp.float32)
    isel_out[...] = jnp.concatenate(
        [i1.astype(jnp.int32), i2.astype(jnp.int32),
         jnp.zeros((BN, NEXP - 2), jnp.int32)], axis=1)
    wsel_out[...] = jnp.concatenate([w1v, w2v, zcol], axis=1)


def _moe_call(h, a0, a1, bcolumn, stats, w1p, rb1, lng, lnb, w2, rb2, cent,
              wr1, br1, wt1, wr2, br2, wt2, interpret=False):
    full = lambda i: (0, 0)
    full3 = lambda i: (0, 0, 0)
    blk = lambda i: (i, 0)
    return pl.pallas_call(
        _moe_body,
        grid=(NB,),
        in_specs=[
            pl.BlockSpec((BN, D), blk),
            pl.BlockSpec((BN, D), blk),
            pl.BlockSpec((BN, D), blk),
            pl.BlockSpec((BN, 1), blk),
            pl.BlockSpec((NG, D), full),
            pl.BlockSpec((136, D), full),
            pl.BlockSpec((1, D), full),
            pl.BlockSpec((1, D), full),
            pl.BlockSpec((1, D), full),
            pl.BlockSpec((D, NEXP), full),
            pl.BlockSpec((1, NEXP), full),
            pl.BlockSpec((1, NEXP), full),
            pl.BlockSpec((NEXP, D, D), full3),
            pl.BlockSpec((NEXP, D), full),
            pl.BlockSpec((NEXP, D, D), full3),
            pl.BlockSpec((NEXP, D, D), full3),
            pl.BlockSpec((NEXP, D), full),
            pl.BlockSpec((NEXP, D, D), full3),
        ],
        out_specs=[
            pl.BlockSpec((NEXP, BN, D), lambda i: (0, i, 0)),
            pl.BlockSpec((BN, D), blk),
            pl.BlockSpec((BN, NEXP), blk),
            pl.BlockSpec((BN, NEXP), blk),
        ],
        out_shape=[
            jax.ShapeDtypeStruct((NEXP, N, D), jnp.float32),
            jax.ShapeDtypeStruct((N, D), jnp.float32),
            jax.ShapeDtypeStruct((N, NEXP), jnp.int32),
            jax.ShapeDtypeStruct((N, NEXP), jnp.float32),
        ],
        interpret=interpret,
    )(h, a0, a1, bcolumn, stats, w1p, rb1, lng, lnb, w2, rb2, cent,
      wr1, br1, wt1, wr2, br2, wt2)


def _comb_body(yp_ref, o1_ref, o2_ref, ws_ref, y_ref):
    ws = ws_ref[...]
    y_ref[...] = (yp_ref[...] + ws[:, 0:1] * o1_ref[...]
                  + ws[:, 1:2] * o2_ref[...])


def _combine_call(yp, o1, o2, ws, interpret=False):
    blk = lambda i: (i, 0)
    return pl.pallas_call(
        _comb_body,
        grid=(NB,),
        in_specs=[
            pl.BlockSpec((BN, D), blk),
            pl.BlockSpec((BN, D), blk),
            pl.BlockSpec((BN, D), blk),
            pl.BlockSpec((BN, NEXP), blk),
        ],
        out_specs=pl.BlockSpec((BN, D), blk),
        out_shape=jax.ShapeDtypeStruct((N, D), jnp.float32),
        interpret=interpret,
    )(yp, o1, o2, ws)


def _zero_acc_slice(accSp, rowsV, base):
    def za(q, carry):
        pltpu.sync_copy(rowsV, accSp.at[pl.ds(base + q * CH, CH)])
        return carry
    lax.fori_loop(0, QF, za, 0)


def _copy_out_slice(accSp, rowsV, base, out_slice_fn):
    def co(q, carry):
        pltpu.sync_copy(accSp.at[pl.ds(base + q * CH, CH)], rowsV)
        pltpu.sync_copy(rowsV, out_slice_fn(base + q * CH, CH))
        return carry
    lax.fori_loop(0, QF, co, 0)


def _pass_a_kernel(interpret=False):
    mesh = plsc.VectorSubcoreMesh(core_axis_name="c", subcore_axis_name="s")

    @functools.partial(
        pl.kernel,
        out_type=jax.ShapeDtypeStruct((NC, NACC, D), jnp.float32),
        mesh=mesh,
        scratch_types=[
            pltpu.VMEM((ROWS_A * CH,), jnp.int32),
            pltpu.VMEM((ROWS_A, CH), jnp.int32),
            pltpu.VMEM((CH, D), jnp.float32),
            pltpu.VMEM_SHARED((NACC, D), jnp.float32),
            pltpu.SemaphoreType.DMA,
        ],
        interpret=interpret,
    )
    def body(h_hbm, srcF, dstR, z_hbm, out_hbm, srcV1, dstV2, rowsV, accSp, sem):
        c = lax.axis_index("c")
        s = lax.axis_index("s")
        w = c * NS + s
        pltpu.sync_copy(srcF.at[pl.ds(w * (ROWS_A * CH), ROWS_A * CH)], srcV1)
        pltpu.sync_copy(dstR.at[w], dstV2)
        pltpu.sync_copy(z_hbm, rowsV)
        base = s * RPT
        _zero_acc_slice(accSp, rowsV, base)
        plsc.subcore_barrier()

        def step(j, carry):
            pltpu.async_copy(h_hbm.at[srcV1.at[pl.ds(j * CH, CH)]],
                             rowsV, sem).wait()
            pltpu.async_copy(rowsV, accSp.at[dstV2.at[j]], sem, add=True).wait()
            return carry
        lax.fori_loop(0, ROWS_A, step, 0)
        plsc.subcore_barrier()
        _copy_out_slice(accSp, rowsV, base,
                        lambda o, l: out_hbm.at[c, pl.ds(o, l)])

    return body


def _pass_bc_kernel(interpret=False):
    mesh = plsc.VectorSubcoreMesh(core_axis_name="c", subcore_axis_name="s")

    @functools.partial(
        pl.kernel,
        out_type=jax.ShapeDtypeStruct((NC, NACC, D), jnp.float32),
        mesh=mesh,
        scratch_types=[
            pltpu.VMEM((HROWS * CH,), jnp.int32),
            pltpu.VMEM((HROWS, CH), jnp.int32),
            pltpu.VMEM((CH, D), jnp.float32),
            pltpu.VMEM((CH,), jnp.int32),
            pltpu.VMEM((CH,), jnp.int32),
            pltpu.VMEM_SHARED((NACC, D), jnp.float32),
            pltpu.SemaphoreType.DMA,
        ],
        interpret=interpret,
    )
    def body(g_hbm, srcF, dstR, isel1_hbm, isel2_hbm, z_hbm, out_hbm,
             srcV1, dstV2, rowsV, idxV, ichV, accSp, sem):
        c = lax.axis_index("c")
        s = lax.axis_index("s")
        pltpu.sync_copy(z_hbm, rowsV)
        base = s * RPT
        _zero_acc_slice(accSp, rowsV, base)
        plsc.subcore_barrier()

        def run_phase(p, carry):
            off = s * (ROWS_BC * CH) + p * (HROWS * CH)
            pltpu.sync_copy(srcF.at[pl.ds(off, HROWS * CH)], srcV1)
            pltpu.sync_copy(dstR.at[s, p], dstV2)

            def step(j, c2):
                @pl.when(c == 0)
                def _():
                    pltpu.async_copy(isel1_hbm.at[dstV2.at[j]], ichV,
                                     sem).wait()

                @pl.when(c == 1)
                def _():
                    pltpu.async_copy(isel2_hbm.at[dstV2.at[j]], ichV,
                                     sem).wait()
                for k in range(CH // 16):
                    s16 = srcV1[pl.ds(j * CH + 16 * k, 16)]
                    i16 = ichV[pl.ds(16 * k, 16)]
                    idxV[pl.ds(16 * k, 16)] = i16 * N + s16
                pltpu.async_copy(g_hbm.at[idxV], rowsV, sem).wait()
                pltpu.async_copy(rowsV, accSp.at[dstV2.at[j]], sem,
                                 add=True).wait()
                return c2
            return lax.fori_loop(0, HROWS, step, carry)
        lax.fori_loop(0, 2, run_phase, 0)
        plsc.subcore_barrier()
        _copy_out_slice(accSp, rowsV, base,
                        lambda o, l: out_hbm.at[c, pl.ds(o, l)])

    return body


def kernel(x, edge_index, batch, enc_W1, enc_b1, enc_W2, enc_b2,
           r_W1, r_b1, r_ln_g, r_ln_b, r_W2, r_b2, size_centers,
           exp_rel_W1, exp_rel_b1, exp_root_W1, exp_rel_W2, exp_rel_b2,
           exp_root_W2):
    src = edge_index[0]
    dst = edge_index[1]
    dstA = dst.reshape(NC * NS, ROWS_A, CH)
    dstB = dst.reshape(NS, 2, HROWS, CH)
    zchunk = jnp.zeros((CH, D), jnp.float32)
    bp2 = jnp.pad(batch, (0, PAD), constant_values=-1).reshape(80, 128)
    src2 = src.reshape(E // 128, 128)
    bcolumn = batch.reshape(N, 1)
    w1p = jnp.pad(r_W1, ((0, 136 - (D + 3)), (0, 0)))
    rb1 = r_b1.reshape(1, D)
    lng = r_ln_g.reshape(1, D)
    lnb = r_ln_b.reshape(1, D)
    rb2 = r_b2.reshape(1, NEXP)
    cent = size_centers.reshape(1, NEXP)

    h = _encoder_call(x, enc_W1, enc_b1.reshape(1, D), enc_W2,
                      enc_b2.reshape(1, D))
    stats = _stats_call(bp2, src2)
    outA = _pass_a_kernel()(h, src, dstA, zchunk)
    g, ypart, isel, wsel = _moe_call(
        h, outA[0], outA[1], bcolumn, stats, w1p, rb1, lng, lnb,
        r_W2, rb2, cent, exp_rel_W1, exp_rel_b1, exp_root_W1,
        exp_rel_W2, exp_rel_b2, exp_root_W2)
    gflat = g.reshape(NEXP * N, D)
    outBC = _pass_bc_kernel()(gflat, src, dstB, isel[:, 0], isel[:, 1],
                              zchunk)
    y = _combine_call(ypart, outBC[0], outBC[1], wsel)
    return y


# speedup vs baseline: 17.0209x; 1.6938x over previous
import functools

import jax
import jax.numpy as jnp
from jax import lax
from jax.experimental import pallas as pl
from jax.experimental.pallas import tpu as pltpu
from jax.experimental.pallas import tpu_sc as plsc

N = 10000
E = 320000
D = 128
NEXP = 8
NG = 16
PAD = 240
NC, NS = 2, 16
CH = 80
NROWS = E // CH
ROWS_A = NROWS // (NC * NS)
ROWS_BC = NROWS // NS
NPH = 5
PH = ROWS_BC // NPH
NB = 10
BN = N // NB
NACC = 10240
RPT = NACC // NS
QF = RPT // CH


def _enc_body(x_ref, w1_ref, b1_ref, w2_ref, b2_ref, h_ref):
    t = jnp.dot(x_ref[...], w1_ref[...], preferred_element_type=jnp.float32)
    t = jnp.maximum(t + b1_ref[...], 0.0)
    h_ref[...] = jnp.dot(t, w2_ref[...], preferred_element_type=jnp.float32) + b2_ref[...]


def _encoder_call(x, w1, b1, w2, b2, interpret=False):
    return pl.pallas_call(
        _enc_body,
        grid=(NB,),
        in_specs=[
            pl.BlockSpec((BN, D), lambda i: (i, 0)),
            pl.BlockSpec((D, D), lambda i: (0, 0)),
            pl.BlockSpec((1, D), lambda i: (0, 0)),
            pl.BlockSpec((D, D), lambda i: (0, 0)),
            pl.BlockSpec((1, D), lambda i: (0, 0)),
        ],
        out_specs=pl.BlockSpec((BN, D), lambda i: (i, 0)),
        out_shape=jax.ShapeDtypeStruct((N, D), jnp.float32),
        interpret=interpret,
    )(x, w1, b1, w2, b2)


def _stats_body(bp_ref, src_ref, stats_ref):
    bp = bp_ref[...]
    src = src_ref[...]
    ns = [jnp.sum((bp == g).astype(jnp.float32)) for g in range(NG)]
    starts = [jnp.sum((bp < g).astype(jnp.int32)) - PAD for g in range(NG + 1)]
    es = []
    for g in range(NG):
        lo, hi = starts[g], starts[g + 1]
        es.append(jnp.sum(((src >= lo) & (src < hi)).astype(jnp.float32)))
    n_col = jnp.concatenate([v.reshape(1, 1) for v in ns], axis=0)
    e_col = jnp.concatenate([v.reshape(1, 1) for v in es], axis=0)
    gmax = jnp.max(bp)
    iota_g = lax.broadcasted_iota(jnp.int32, (NG, 1), 0)
    valid = iota_g <= gmax
    gcount = (gmax + 1).astype(jnp.float32)
    n_c = jnp.clip(n_col, 1.0, None)
    density = e_col / jnp.clip(n_c * (n_c - 1.0), 1.0, None)
    logn = jnp.log(n_c)
    logn_min = jnp.min(jnp.where(valid, logn, jnp.inf))
    logn_max = jnp.max(jnp.where(valid, logn, -jnp.inf))
    logn_norm = (logn - logn_min) / (logn_max - logn_min + 1e-6)
    feats = jnp.concatenate([n_c, e_col, density], axis=1)
    fmean = jnp.sum(jnp.where(valid, feats, 0.0), axis=0, keepdims=True) / gcount
    fvar = jnp.sum(jnp.where(valid, (feats - fmean) ** 2, 0.0), axis=0, keepdims=True) / gcount
    feats = (feats - fmean) / (jnp.sqrt(fvar) + 1e-6)
    stats_ref[...] = jnp.concatenate(
        [feats, logn_norm, jnp.zeros((NG, D - 4), jnp.float32)], axis=1)


def _stats_call(bp2, src2, interpret=False):
    return pl.pallas_call(
        _stats_body,
        out_shape=jax.ShapeDtypeStruct((NG, D), jnp.float32),
        interpret=interpret,
    )(bp2, src2)


def _moe_body(h_ref, a0_ref, a1_ref, bc_ref, st_ref, w1p_ref, rb1_ref,
              lng_ref, lnb_ref, w2_ref, rb2_ref, cent_ref,
              wr1_ref, br1_ref, wt1_ref, wr2_ref, br2_ref, wt2_ref,
              g_out, yp_out, isel_out, wsel_out):
    h = h_ref[...]
    agg1 = a0_ref[...] + a1_ref[...]
    bcol = bc_ref[...]
    iota_g = lax.broadcasted_iota(jnp.int32, (1, NG), 1)
    P = (bcol == iota_g).astype(jnp.float32)
    st = st_ref[...]
    feats3 = st[:, 0:3]
    logn16 = st[:, 3:4]
    w1p = w1p_ref[...]
    sf_w = jnp.dot(feats3, w1p[128:131, :], preferred_element_type=jnp.float32)
    t = (jnp.dot(h, w1p[0:128, :], preferred_element_type=jnp.float32)
         + jnp.dot(P, sf_w, preferred_element_type=jnp.float32,
                   precision=lax.Precision.HIGHEST) + rb1_ref[...])
    mu = jnp.mean(t, axis=1, keepdims=True)
    var = jnp.mean((t - mu) ** 2, axis=1, keepdims=True)
    tn = (t - mu) / jnp.sqrt(var + 1e-5) * lng_ref[...] + lnb_ref[...]
    learned = jnp.dot(jnp.maximum(tn, 0.0), w2_ref[...],
                      preferred_element_type=jnp.float32) + rb2_ref[...]
    logn_n = jnp.dot(P, logn16, preferred_element_type=jnp.float32,
                     precision=lax.Precision.HIGHEST)
    prior = -(logn_n - cent_ref[...]) ** 2
    logits = 0.7 * learned + 0.3 * prior
    z = logits - jnp.max(logits, axis=1, keepdims=True)
    pz = jnp.exp(z)
    probs = pz / jnp.sum(pz, axis=1, keepdims=True)
    io8 = lax.broadcasted_iota(jnp.int32, (1, NEXP), 1).astype(jnp.float32)
    m1 = jnp.max(probs, axis=1, keepdims=True)
    i1 = jnp.min(jnp.where(probs == m1, io8, 99.0), axis=1, keepdims=True)
    p2 = jnp.where(io8 == i1, -1.0, probs)
    m2 = jnp.max(p2, axis=1, keepdims=True)
    i2 = jnp.min(jnp.where(p2 == m2, io8, 99.0), axis=1, keepdims=True)
    s = m1 + m2 + 1e-8
    w1v = m1 / s
    w2v = m2 / s
    yp = jnp.zeros((BN, D), jnp.float32)
    for i in range(NEXP):
        h1 = jnp.maximum(
            jnp.dot(agg1, wr1_ref[i], preferred_element_type=jnp.float32)
            + br1_ref[i:i + 1, :]
            + jnp.dot(h, wt1_ref[i], preferred_element_type=jnp.float32), 0.0)
        g_out[i] = jnp.dot(h1, wr2_ref[i], preferred_element_type=jnp.float32)
        r2 = jnp.dot(h1, wt2_ref[i], preferred_element_type=jnp.float32)
        coef = (w1v * (i1 == float(i)).astype(jnp.float32)
                + w2v * (i2 == float(i)).astype(jnp.float32))
        yp = yp + coef * (r2 + br2_ref[i:i + 1, :])
    yp_out[...] = yp
    zcol = jnp.zeros((BN, NEXP - 2), jnp.float32)
    isel_out[...] = jnp.concatenate(
        [i1.astype(jnp.int32), i2.astype(jnp.int32),
         jnp.zeros((BN, NEXP - 2), jnp.int32)], axis=1)
    wsel_out[...] = jnp.concatenate([w1v, w2v, zcol], axis=1)


def _moe_call(h, a0, a1, bcolumn, stats, w1p, rb1, lng, lnb, w2, rb2, cent,
              wr1, br1, wt1, wr2, br2, wt2, interpret=False):
    full = lambda i: (0, 0)
    full3 = lambda i: (0, 0, 0)
    blk = lambda i: (i, 0)
    return pl.pallas_call(
        _moe_body,
        grid=(NB,),
        in_specs=[
            pl.BlockSpec((BN, D), blk),
            pl.BlockSpec((BN, D), blk),
            pl.BlockSpec((BN, D), blk),
            pl.BlockSpec((BN, 1), blk),
            pl.BlockSpec((NG, D), full),
            pl.BlockSpec((136, D), full),
            pl.BlockSpec((1, D), full),
            pl.BlockSpec((1, D), full),
            pl.BlockSpec((1, D), full),
            pl.BlockSpec((D, NEXP), full),
            pl.BlockSpec((1, NEXP), full),
            pl.BlockSpec((1, NEXP), full),
            pl.BlockSpec((NEXP, D, D), full3),
            pl.BlockSpec((NEXP, D), full),
            pl.BlockSpec((NEXP, D, D), full3),
            pl.BlockSpec((NEXP, D, D), full3),
            pl.BlockSpec((NEXP, D), full),
            pl.BlockSpec((NEXP, D, D), full3),
        ],
        out_specs=[
            pl.BlockSpec((NEXP, BN, D), lambda i: (0, i, 0)),
            pl.BlockSpec((BN, D), blk),
            pl.BlockSpec((BN, NEXP), blk),
            pl.BlockSpec((BN, NEXP), blk),
        ],
        out_shape=[
            jax.ShapeDtypeStruct((NEXP, N, D), jnp.float32),
            jax.ShapeDtypeStruct((N, D), jnp.float32),
            jax.ShapeDtypeStruct((N, NEXP), jnp.int32),
            jax.ShapeDtypeStruct((N, NEXP), jnp.float32),
        ],
        interpret=interpret,
    )(h, a0, a1, bcolumn, stats, w1p, rb1, lng, lnb, w2, rb2, cent,
      wr1, br1, wt1, wr2, br2, wt2)


def _comb_body(yp_ref, o1_ref, o2_ref, ws_ref, y_ref):
    ws = ws_ref[...]
    y_ref[...] = (yp_ref[...] + ws[:, 0:1] * o1_ref[...]
                  + ws[:, 1:2] * o2_ref[...])


def _combine_call(yp, o1, o2, ws, interpret=False):
    blk = lambda i: (i, 0)
    return pl.pallas_call(
        _comb_body,
        grid=(NB,),
        in_specs=[
            pl.BlockSpec((BN, D), blk),
            pl.BlockSpec((BN, D), blk),
            pl.BlockSpec((BN, D), blk),
            pl.BlockSpec((BN, NEXP), blk),
        ],
        out_specs=pl.BlockSpec((BN, D), blk),
        out_shape=jax.ShapeDtypeStruct((N, D), jnp.float32),
        interpret=interpret,
    )(yp, o1, o2, ws)


def _zero_acc_slice(accSp, rowsV, base):
    def za(q, carry):
        pltpu.sync_copy(rowsV, accSp.at[pl.ds(base + q * CH, CH)])
        return carry
    lax.fori_loop(0, QF, za, 0)


def _copy_out_slice(accSp, rowsV, base, out_slice_fn):
    def co(q, carry):
        pltpu.sync_copy(accSp.at[pl.ds(base + q * CH, CH)], rowsV)
        pltpu.sync_copy(rowsV, out_slice_fn(base + q * CH, CH))
        return carry
    lax.fori_loop(0, QF, co, 0)


def _pass_a_kernel(interpret=False):
    mesh = plsc.VectorSubcoreMesh(core_axis_name="c", subcore_axis_name="s")

    @functools.partial(
        pl.kernel,
        out_type=jax.ShapeDtypeStruct((NC, NACC, D), jnp.float32),
        mesh=mesh,
        scratch_types=[
            pltpu.VMEM((ROWS_A * CH,), jnp.int32),
            pltpu.VMEM((ROWS_A, CH), jnp.int32),
            pltpu.VMEM((2, CH, D), jnp.float32),
            pltpu.VMEM_SHARED((NACC, D), jnp.float32),
            pltpu.SemaphoreType.DMA,
            pltpu.SemaphoreType.DMA,
            pltpu.SemaphoreType.DMA,
            pltpu.SemaphoreType.DMA,
        ],
        interpret=interpret,
    )
    def body(h_hbm, srcF, dstR, z_hbm, out_hbm, srcV1, dstV2, rows2, accSp,
             g0, g1, s0, s1):
        c = lax.axis_index("c")
        s = lax.axis_index("s")
        w = c * NS + s
        pltpu.sync_copy(srcF.at[pl.ds(w * (ROWS_A * CH), ROWS_A * CH)], srcV1)
        pltpu.sync_copy(dstR.at[w], dstV2)
        pltpu.sync_copy(z_hbm, rows2.at[0])
        base = s * RPT
        _zero_acc_slice(accSp, rows2.at[0], base)
        plsc.subcore_barrier()

        gsem = (g0, g1)
        ssem = (s0, s1)

        def issue_gather(j, b):
            pltpu.async_copy(h_hbm.at[srcV1.at[pl.ds(j * CH, CH)]],
                             rows2.at[b], gsem[b])

        def issue_scatter(j, b):
            pltpu.async_copy(rows2.at[b], accSp.at[dstV2.at[j]], ssem[b],
                             add=True)

        def drain(b, sems):
            pltpu.make_async_copy(h_hbm.at[pl.ds(0, CH)], rows2.at[b],
                                  sems[b]).wait()

        issue_gather(0, 0)
        issue_gather(1, 1)

        def pair(jj, carry):
            j0 = 2 * jj
            drain(0, gsem)
            issue_scatter(j0, 0)
            drain(0, ssem)
            issue_gather(j0 + 2, 0)
            drain(1, gsem)
            issue_scatter(j0 + 1, 1)

            @pl.when(jj < (ROWS_A - 1) // 2 - 1)
            def _():
                drain(1, ssem)
                issue_gather(j0 + 3, 1)
            return carry
        lax.fori_loop(0, (ROWS_A - 1) // 2, pair, 0)
        drain(0, gsem)
        issue_scatter(ROWS_A - 1, 0)
        drain(0, ssem)
        drain(1, ssem)
        plsc.subcore_barrier()
        _copy_out_slice(accSp, rows2.at[0], base,
                        lambda o, l: out_hbm.at[c, pl.ds(o, l)])

    return body


def _pass_bc_kernel(interpret=False):
    mesh = plsc.VectorSubcoreMesh(core_axis_name="c", subcore_axis_name="s")

    @functools.partial(
        pl.kernel,
        out_type=jax.ShapeDtypeStruct((NC, NACC, D), jnp.float32),
        mesh=mesh,
        scratch_types=[
            pltpu.VMEM((PH * CH,), jnp.int32),
            pltpu.VMEM((PH, CH), jnp.int32),
            pltpu.VMEM((PH * CH,), jnp.int32),
            pltpu.VMEM((2, CH, D), jnp.float32),
            pltpu.VMEM_SHARED((NACC, D), jnp.float32),
            pltpu.SemaphoreType.DMA,
            pltpu.SemaphoreType.DMA,
            pltpu.SemaphoreType.DMA,
            pltpu.SemaphoreType.DMA,
            pltpu.SemaphoreType.DMA,
        ],
        interpret=interpret,
    )
    def body(g_hbm, srcF, dstR, isel1_hbm, isel2_hbm, z_hbm, out_hbm,
             srcV1, dstV2, ichV, rows2, accSp, g0, g1, s0, s1, isem):
        c = lax.axis_index("c")
        s = lax.axis_index("s")
        pltpu.sync_copy(z_hbm, rows2.at[0])
        base = s * RPT
        _zero_acc_slice(accSp, rows2.at[0], base)
        plsc.subcore_barrier()

        gsem = (g0, g1)
        ssem = (s0, s1)

        def issue_gather(j, b):
            pltpu.async_copy(g_hbm.at[srcV1.at[pl.ds(j * CH, CH)]],
                             rows2.at[b], gsem[b])

        def issue_scatter(j, b):
            pltpu.async_copy(rows2.at[b], accSp.at[dstV2.at[j]], ssem[b],
                             add=True)

        def drain(b, sems):
            pltpu.make_async_copy(g_hbm.at[pl.ds(0, CH)], rows2.at[b],
                                  sems[b]).wait()

        def run_phase(p, carry):
            off = s * (ROWS_BC * CH) + p * (PH * CH)
            pltpu.sync_copy(srcF.at[pl.ds(off, PH * CH)], srcV1)
            pltpu.sync_copy(dstR.at[s, p], dstV2)

            @pl.when(c == 0)
            def _():
                def fire1(q, c2):
                    pltpu.async_copy(isel1_hbm.at[dstV2.at[q]],
                                     ichV.at[pl.ds(q * CH, CH)], isem)
                    return c2
                lax.fori_loop(0, PH, fire1, 0)

            @pl.when(c == 1)
            def _():
                def fire2(q, c2):
                    pltpu.async_copy(isel2_hbm.at[dstV2.at[q]],
                                     ichV.at[pl.ds(q * CH, CH)], isem)
                    return c2
                lax.fori_loop(0, PH, fire2, 0)

            def dfire(q, c2):
                pltpu.make_async_copy(isel1_hbm.at[pl.ds(0, CH)],
                                      ichV.at[pl.ds(0, CH)], isem).wait()
                return c2
            lax.fori_loop(0, PH, dfire, 0)

            def cidx(i, c2):
                sl = pl.ds(i * 16, 16)
                srcV1[sl] = ichV[sl] * N + srcV1[sl]
                return c2
            lax.fori_loop(0, PH * CH // 16, cidx, 0)

            issue_gather(0, 0)
            issue_gather(1, 1)

            def pair(jj, c2):
                j0 = 2 * jj
                drain(0, gsem)
                issue_scatter(j0, 0)

                @pl.when(jj < PH // 2 - 1)
                def _():
                    drain(0, ssem)
                    issue_gather(j0 + 2, 0)
                drain(1, gsem)
                issue_scatter(j0 + 1, 1)

                @pl.when(jj < PH // 2 - 1)
                def _():
                    drain(1, ssem)
                    issue_gather(j0 + 3, 1)
                return c2
            lax.fori_loop(0, PH // 2, pair, 0)
            drain(0, ssem)
            drain(1, ssem)
            return carry
        lax.fori_loop(0, NPH, run_phase, 0)
        plsc.subcore_barrier()
        _copy_out_slice(accSp, rows2.at[0], base,
                        lambda o, l: out_hbm.at[c, pl.ds(o, l)])

    return body


def kernel(x, edge_index, batch, enc_W1, enc_b1, enc_W2, enc_b2,
           r_W1, r_b1, r_ln_g, r_ln_b, r_W2, r_b2, size_centers,
           exp_rel_W1, exp_rel_b1, exp_root_W1, exp_rel_W2, exp_rel_b2,
           exp_root_W2):
    src = edge_index[0]
    dst = edge_index[1]
    dstA = dst.reshape(NC * NS, ROWS_A, CH)
    dstB = dst.reshape(NS, NPH, PH, CH)
    zchunk = jnp.zeros((CH, D), jnp.float32)
    bp2 = jnp.pad(batch, (0, PAD), constant_values=-1).reshape(80, 128)
    src2 = src.reshape(E // 128, 128)
    bcolumn = batch.reshape(N, 1)
    w1p = jnp.pad(r_W1, ((0, 136 - (D + 3)), (0, 0)))
    rb1 = r_b1.reshape(1, D)
    lng = r_ln_g.reshape(1, D)
    lnb = r_ln_b.reshape(1, D)
    rb2 = r_b2.reshape(1, NEXP)
    cent = size_centers.reshape(1, NEXP)

    h = _encoder_call(x, enc_W1, enc_b1.reshape(1, D), enc_W2,
                      enc_b2.reshape(1, D))
    stats = _stats_call(bp2, src2)
    outA = _pass_a_kernel()(h, src, dstA, zchunk)
    g, ypart, isel, wsel = _moe_call(
        h, outA[0], outA[1], bcolumn, stats, w1p, rb1, lng, lnb,
        r_W2, rb2, cent, exp_rel_W1, exp_rel_b1, exp_root_W1,
        exp_rel_W2, exp_rel_b2, exp_root_W2)
    gflat = g.reshape(NEXP * N, D)
    outBC = _pass_bc_kernel()(gflat, src, dstB, isel[:, 0], isel[:, 1],
                              zchunk)
    y = _combine_call(ypart, outBC[0], outBC[1], wsel)
    return y


# speedup vs baseline: 18.6303x; 1.0946x over previous
import functools

import jax
import jax.numpy as jnp
from jax import lax
from jax.experimental import pallas as pl
from jax.experimental.pallas import tpu as pltpu
from jax.experimental.pallas import tpu_sc as plsc

N = 10000
E = 320000
D = 128
NEXP = 8
NG = 16
PAD = 240
NC, NS = 2, 16
CH = 80
NBUF_A = 4
NBUF_BC = 3
NROWS = E // CH
ROWS_A = NROWS // (NC * NS)
ROWS_BC = NROWS // NS
NPH_A = 5
PH_A = ROWS_A // NPH_A
NPH = 10
PH = ROWS_BC // NPH
NB = 10
BN = N // NB
NACC = 10240
RPT = NACC // NS
QF = RPT // CH


def _enc_body(x_ref, w1_ref, b1_ref, w2_ref, b2_ref, h_ref):
    t = jnp.dot(x_ref[...], w1_ref[...], preferred_element_type=jnp.float32)
    t = jnp.maximum(t + b1_ref[...], 0.0)
    h_ref[...] = jnp.dot(t, w2_ref[...], preferred_element_type=jnp.float32) + b2_ref[...]


def _encoder_call(x, w1, b1, w2, b2, interpret=False):
    return pl.pallas_call(
        _enc_body,
        grid=(NB,),
        in_specs=[
            pl.BlockSpec((BN, D), lambda i: (i, 0)),
            pl.BlockSpec((D, D), lambda i: (0, 0)),
            pl.BlockSpec((1, D), lambda i: (0, 0)),
            pl.BlockSpec((D, D), lambda i: (0, 0)),
            pl.BlockSpec((1, D), lambda i: (0, 0)),
        ],
        out_specs=pl.BlockSpec((BN, D), lambda i: (i, 0)),
        out_shape=jax.ShapeDtypeStruct((N, D), jnp.float32),
        interpret=interpret,
    )(x, w1, b1, w2, b2)


def _stats_body(bp_ref, src_ref, stats_ref):
    bp = bp_ref[...]
    src = src_ref[...]
    ns = [jnp.sum((bp == g).astype(jnp.float32)) for g in range(NG)]
    starts = [jnp.sum((bp < g).astype(jnp.int32)) - PAD for g in range(NG + 1)]
    es = []
    for g in range(NG):
        lo, hi = starts[g], starts[g + 1]
        es.append(jnp.sum(((src >= lo) & (src < hi)).astype(jnp.float32)))
    n_col = jnp.concatenate([v.reshape(1, 1) for v in ns], axis=0)
    e_col = jnp.concatenate([v.reshape(1, 1) for v in es], axis=0)
    gmax = jnp.max(bp)
    iota_g = lax.broadcasted_iota(jnp.int32, (NG, 1), 0)
    valid = iota_g <= gmax
    gcount = (gmax + 1).astype(jnp.float32)
    n_c = jnp.clip(n_col, 1.0, None)
    density = e_col / jnp.clip(n_c * (n_c - 1.0), 1.0, None)
    logn = jnp.log(n_c)
    logn_min = jnp.min(jnp.where(valid, logn, jnp.inf))
    logn_max = jnp.max(jnp.where(valid, logn, -jnp.inf))
    logn_norm = (logn - logn_min) / (logn_max - logn_min + 1e-6)
    feats = jnp.concatenate([n_c, e_col, density], axis=1)
    fmean = jnp.sum(jnp.where(valid, feats, 0.0), axis=0, keepdims=True) / gcount
    fvar = jnp.sum(jnp.where(valid, (feats - fmean) ** 2, 0.0), axis=0, keepdims=True) / gcount
    feats = (feats - fmean) / (jnp.sqrt(fvar) + 1e-6)
    stats_ref[...] = jnp.concatenate(
        [feats, logn_norm, jnp.zeros((NG, D - 4), jnp.float32)], axis=1)


def _stats_call(bp2, src2, interpret=False):
    return pl.pallas_call(
        _stats_body,
        out_shape=jax.ShapeDtypeStruct((NG, D), jnp.float32),
        interpret=interpret,
    )(bp2, src2)


def _moe_body(h_ref, a0_ref, a1_ref, bc_ref, st_ref, w1p_ref, rb1_ref,
              lng_ref, lnb_ref, w2_ref, rb2_ref, cent_ref,
              wr1_ref, br1_ref, wt1_ref, wr2_ref, br2_ref, wt2_ref,
              g_out, yp_out, isel_out, wsel_out):
    h = h_ref[...]
    agg1 = a0_ref[...] + a1_ref[...]
    bcol = bc_ref[...]
    iota_g = lax.broadcasted_iota(jnp.int32, (1, NG), 1)
    P = (bcol == iota_g).astype(jnp.float32)
    st = st_ref[...]
    feats3 = st[:, 0:3]
    logn16 = st[:, 3:4]
    w1p = w1p_ref[...]
    sf_w = jnp.dot(feats3, w1p[128:131, :], preferred_element_type=jnp.float32)
    t = (jnp.dot(h, w1p[0:128, :], preferred_element_type=jnp.float32)
         + jnp.dot(P, sf_w, preferred_element_type=jnp.float32,
                   precision=lax.Precision.HIGHEST) + rb1_ref[...])
    mu = jnp.mean(t, axis=1, keepdims=True)
    var = jnp.mean((t - mu) ** 2, axis=1, keepdims=True)
    tn = (t - mu) / jnp.sqrt(var + 1e-5) * lng_ref[...] + lnb_ref[...]
    learned = jnp.dot(jnp.maximum(tn, 0.0), w2_ref[...],
                      preferred_element_type=jnp.float32) + rb2_ref[...]
    logn_n = jnp.dot(P, logn16, preferred_element_type=jnp.float32,
                     precision=lax.Precision.HIGHEST)
    prior = -(logn_n - cent_ref[...]) ** 2
    logits = 0.7 * learned + 0.3 * prior
    z = logits - jnp.max(logits, axis=1, keepdims=True)
    pz = jnp.exp(z)
    probs = pz / jnp.sum(pz, axis=1, keepdims=True)
    io8 = lax.broadcasted_iota(jnp.int32, (1, NEXP), 1).astype(jnp.float32)
    m1 = jnp.max(probs, axis=1, keepdims=True)
    i1 = jnp.min(jnp.where(probs == m1, io8, 99.0), axis=1, keepdims=True)
    p2 = jnp.where(io8 == i1, -1.0, probs)
    m2 = jnp.max(p2, axis=1, keepdims=True)
    i2 = jnp.min(jnp.where(p2 == m2, io8, 99.0), axis=1, keepdims=True)
    s = m1 + m2 + 1e-8
    w1v = m1 / s
    w2v = m2 / s
    yp = jnp.zeros((BN, D), jnp.float32)
    for i in range(NEXP):
        h1 = jnp.maximum(
            jnp.dot(agg1, wr1_ref[i], preferred_element_type=jnp.float32)
            + br1_ref[i:i + 1, :]
            + jnp.dot(h, wt1_ref[i], preferred_element_type=jnp.float32), 0.0)
        g_out[i] = jnp.dot(h1, wr2_ref[i], preferred_element_type=jnp.float32)
        r2 = jnp.dot(h1, wt2_ref[i], preferred_element_type=jnp.float32)
        coef = (w1v * (i1 == float(i)).astype(jnp.float32)
                + w2v * (i2 == float(i)).astype(jnp.float32))
        yp = yp + coef * (r2 + br2_ref[i:i + 1, :])
    yp_out[...] = yp
    zcol = jnp.zeros((BN, NEXP - 2), jnp.float32)
    isel_out[...] = jnp.concatenate(
        [i1.astype(jnp.int32), i2.astype(jnp.int32),
         jnp.zeros((BN, NEXP - 2), jnp.int32)], axis=1)
    wsel_out[...] = jnp.concatenate([w1v, w2v, zcol], axis=1)


def _moe_call(h, a0, a1, bcolumn, stats, w1p, rb1, lng, lnb, w2, rb2, cent,
              wr1, br1, wt1, wr2, br2, wt2, interpret=False):
    full = lambda i: (0, 0)
    full3 = lambda i: (0, 0, 0)
    blk = lambda i: (i, 0)
    return pl.pallas_call(
        _moe_body,
        grid=(NB,),
        in_specs=[
            pl.BlockSpec((BN, D), blk),
            pl.BlockSpec((BN, D), blk),
            pl.BlockSpec((BN, D), blk),
            pl.BlockSpec((BN, 1), blk),
            pl.BlockSpec((NG, D), full),
            pl.BlockSpec((136, D), full),
            pl.BlockSpec((1, D), full),
            pl.BlockSpec((1, D), full),
            pl.BlockSpec((1, D), full),
            pl.BlockSpec((D, NEXP), full),
            pl.BlockSpec((1, NEXP), full),
            pl.BlockSpec((1, NEXP), full),
            pl.BlockSpec((NEXP, D, D), full3),
            pl.BlockSpec((NEXP, D), full),
            pl.BlockSpec((NEXP, D, D), full3),
            pl.BlockSpec((NEXP, D, D), full3),
            pl.BlockSpec((NEXP, D), full),
            pl.BlockSpec((NEXP, D, D), full3),
        ],
        out_specs=[
            pl.BlockSpec((NEXP, BN, D), lambda i: (0, i, 0)),
            pl.BlockSpec((BN, D), blk),
            pl.BlockSpec((BN, NEXP), blk),
            pl.BlockSpec((BN, NEXP), blk),
        ],
        out_shape=[
            jax.ShapeDtypeStruct((NEXP, N, D), jnp.float32),
            jax.ShapeDtypeStruct((N, D), jnp.float32),
            jax.ShapeDtypeStruct((N, NEXP), jnp.int32),
            jax.ShapeDtypeStruct((N, NEXP), jnp.float32),
        ],
        interpret=interpret,
    )(h, a0, a1, bcolumn, stats, w1p, rb1, lng, lnb, w2, rb2, cent,
      wr1, br1, wt1, wr2, br2, wt2)


def _comb_body(yp_ref, o1_ref, o2_ref, ws_ref, y_ref):
    ws = ws_ref[...]
    y_ref[...] = (yp_ref[...] + ws[:, 0:1] * o1_ref[...]
                  + ws[:, 1:2] * o2_ref[...])


def _combine_call(yp, o1, o2, ws, interpret=False):
    blk = lambda i: (i, 0)
    return pl.pallas_call(
        _comb_body,
        grid=(NB,),
        in_specs=[
            pl.BlockSpec((BN, D), blk),
            pl.BlockSpec((BN, D), blk),
            pl.BlockSpec((BN, D), blk),
            pl.BlockSpec((BN, NEXP), blk),
        ],
        out_specs=pl.BlockSpec((BN, D), blk),
        out_shape=jax.ShapeDtypeStruct((N, D), jnp.float32),
        interpret=interpret,
    )(yp, o1, o2, ws)


def _zero_acc_slice(accSp, rowsV, base):
    def za(q, carry):
        pltpu.sync_copy(rowsV, accSp.at[pl.ds(base + q * CH, CH)])
        return carry
    lax.fori_loop(0, QF, za, 0)


def _copy_out_slice(accSp, rowsV, base, out_slice_fn):
    def co(q, carry):
        pltpu.sync_copy(accSp.at[pl.ds(base + q * CH, CH)], rowsV)
        pltpu.sync_copy(rowsV, out_slice_fn(base + q * CH, CH))
        return carry
    lax.fori_loop(0, QF, co, 0)


def _pipelined_chunks(n, nbuf, issue_gather, issue_scatter, drain_g, drain_s):
    for b in range(nbuf):
        issue_gather(b, b)

    def group(gg, carry):
        j0 = gg * nbuf
        for b in range(nbuf):
            j = j0 + b

            @pl.when(j < n)
            def _(j=j, b=b):
                drain_g(b)
                issue_scatter(j, b)

                @pl.when(j + nbuf < n)
                def _():
                    drain_s(b)
                    issue_gather(j + nbuf, b)
        return carry
    lax.fori_loop(0, -(-n // nbuf), group, 0)
    for b in range(nbuf):
        drain_s(b)


def _pass_a_kernel(interpret=False):
    mesh = plsc.VectorSubcoreMesh(core_axis_name="c", subcore_axis_name="s")

    @functools.partial(
        pl.kernel,
        out_type=jax.ShapeDtypeStruct((NC, NACC, D), jnp.float32),
        mesh=mesh,
        scratch_types=[
            pltpu.VMEM((PH_A * CH,), jnp.int32),
            pltpu.VMEM((PH_A, CH), jnp.int32),
            pltpu.VMEM((NBUF_A, CH, D), jnp.float32),
            pltpu.VMEM_SHARED((NACC, D), jnp.float32),
        ] + [pltpu.SemaphoreType.DMA] * (2 * NBUF_A),
        interpret=interpret,
    )
    def body(h_hbm, srcF, dstR, z_hbm, out_hbm, srcV1, dstV2, rowsB, accSp,
             *sems):
        gsem = sems[:NBUF_A]
        ssem = sems[NBUF_A:]
        c = lax.axis_index("c")
        s = lax.axis_index("s")
        w = c * NS + s
        pltpu.sync_copy(z_hbm, rowsB.at[0])
        base = s * RPT
        _zero_acc_slice(accSp, rowsB.at[0], base)
        plsc.subcore_barrier()

        def issue_gather(j, b):
            pltpu.async_copy(h_hbm.at[srcV1.at[pl.ds(j * CH, CH)]],
                             rowsB.at[b], gsem[b])

        def issue_scatter(j, b):
            pltpu.async_copy(rowsB.at[b], accSp.at[dstV2.at[j]], ssem[b],
                             add=True)

        def drain_g(b):
            pltpu.make_async_copy(h_hbm.at[pl.ds(0, CH)], rowsB.at[b],
                                  gsem[b]).wait()

        def drain_s(b):
            pltpu.make_async_copy(h_hbm.at[pl.ds(0, CH)], rowsB.at[b],
                                  ssem[b]).wait()

        def run_phase(p, carry):
            off = w * (ROWS_A * CH) + p * (PH_A * CH)
            pltpu.sync_copy(srcF.at[pl.ds(off, PH_A * CH)], srcV1)
            pltpu.sync_copy(dstR.at[w, p], dstV2)
            _pipelined_chunks(PH_A, NBUF_A, issue_gather, issue_scatter,
                              drain_g, drain_s)
            return carry
        lax.fori_loop(0, NPH_A, run_phase, 0)
        plsc.subcore_barrier()
        _copy_out_slice(accSp, rowsB.at[0], base,
                        lambda o, l: out_hbm.at[c, pl.ds(o, l)])

    return body


def _pass_bc_kernel(interpret=False):
    mesh = plsc.VectorSubcoreMesh(core_axis_name="c", subcore_axis_name="s")

    @functools.partial(
        pl.kernel,
        out_type=jax.ShapeDtypeStruct((NC, NACC, D), jnp.float32),
        mesh=mesh,
        scratch_types=[
            pltpu.VMEM((PH * CH,), jnp.int32),
            pltpu.VMEM((PH, CH), jnp.int32),
            pltpu.VMEM((PH * CH,), jnp.int32),
            pltpu.VMEM((NBUF_BC, CH, D), jnp.float32),
            pltpu.VMEM_SHARED((NACC, D), jnp.float32),
        ] + [pltpu.SemaphoreType.DMA] * (2 * NBUF_BC + 1),
        interpret=interpret,
    )
    def body(g_hbm, srcF, dstR, isel1_hbm, isel2_hbm, z_hbm, out_hbm,
             srcV1, dstV2, ichV, rowsB, accSp, *sems):
        gsem = sems[:NBUF_BC]
        ssem = sems[NBUF_BC:2 * NBUF_BC]
        isem = sems[2 * NBUF_BC]
        c = lax.axis_index("c")
        s = lax.axis_index("s")
        pltpu.sync_copy(z_hbm, rowsB.at[0])
        base = s * RPT
        _zero_acc_slice(accSp, rowsB.at[0], base)
        plsc.subcore_barrier()

        def issue_gather(j, b):
            pltpu.async_copy(g_hbm.at[srcV1.at[pl.ds(j * CH, CH)]],
                             rowsB.at[b], gsem[b])

        def issue_scatter(j, b):
            pltpu.async_copy(rowsB.at[b], accSp.at[dstV2.at[j]], ssem[b],
                             add=True)

        def drain_g(b):
            pltpu.make_async_copy(g_hbm.at[pl.ds(0, CH)], rowsB.at[b],
                                  gsem[b]).wait()

        def drain_s(b):
            pltpu.make_async_copy(g_hbm.at[pl.ds(0, CH)], rowsB.at[b],
                                  ssem[b]).wait()

        def run_phase(p, carry):
            off = s * (ROWS_BC * CH) + p * (PH * CH)
            pltpu.sync_copy(srcF.at[pl.ds(off, PH * CH)], srcV1)
            pltpu.sync_copy(dstR.at[s, p], dstV2)

            @pl.when(c == 0)
            def _():
                def fire1(q, c2):
                    pltpu.async_copy(isel1_hbm.at[dstV2.at[q]],
                                     ichV.at[pl.ds(q * CH, CH)], isem)
                    return c2
                lax.fori_loop(0, PH, fire1, 0)

            @pl.when(c == 1)
            def _():
                def fire2(q, c2):
                    pltpu.async_copy(isel2_hbm.at[dstV2.at[q]],
                                     ichV.at[pl.ds(q * CH, CH)], isem)
                    return c2
                lax.fori_loop(0, PH, fire2, 0)

            def dfire(q, c2):
                pltpu.make_async_copy(isel1_hbm.at[pl.ds(0, CH)],
                                      ichV.at[pl.ds(0, CH)], isem).wait()
                return c2
            lax.fori_loop(0, PH, dfire, 0)

            def cidx(i, c2):
                sl = pl.ds(i * 16, 16)
                srcV1[sl] = ichV[sl] * N + srcV1[sl]
                return c2
            lax.fori_loop(0, PH * CH // 16, cidx, 0)

            _pipelined_chunks(PH, NBUF_BC, issue_gather, issue_scatter,
                              drain_g, drain_s)
            return carry
        lax.fori_loop(0, NPH, run_phase, 0)
        plsc.subcore_barrier()
        _copy_out_slice(accSp, rowsB.at[0], base,
                        lambda o, l: out_hbm.at[c, pl.ds(o, l)])

    return body


def kernel(x, edge_index, batch, enc_W1, enc_b1, enc_W2, enc_b2,
           r_W1, r_b1, r_ln_g, r_ln_b, r_W2, r_b2, size_centers,
           exp_rel_W1, exp_rel_b1, exp_root_W1, exp_rel_W2, exp_rel_b2,
           exp_root_W2):
    src = edge_index[0]
    dst = edge_index[1]
    dstA = dst.reshape(NC * NS, NPH_A, PH_A, CH)
    dstB = dst.reshape(NS, NPH, PH, CH)
    zchunk = jnp.zeros((CH, D), jnp.float32)
    bp2 = jnp.pad(batch, (0, PAD), constant_values=-1).reshape(80, 128)
    src2 = src.reshape(E // 128, 128)
    bcolumn = batch.reshape(N, 1)
    w1p = jnp.pad(r_W1, ((0, 136 - (D + 3)), (0, 0)))
    rb1 = r_b1.reshape(1, D)
    lng = r_ln_g.reshape(1, D)
    lnb = r_ln_b.reshape(1, D)
    rb2 = r_b2.reshape(1, NEXP)
    cent = size_centers.reshape(1, NEXP)

    h = _encoder_call(x, enc_W1, enc_b1.reshape(1, D), enc_W2,
                      enc_b2.reshape(1, D))
    stats = _stats_call(bp2, src2)
    outA = _pass_a_kernel()(h, src, dstA, zchunk)
    g, ypart, isel, wsel = _moe_call(
        h, outA[0], outA[1], bcolumn, stats, w1p, rb1, lng, lnb,
        r_W2, rb2, cent, exp_rel_W1, exp_rel_b1, exp_root_W1,
        exp_rel_W2, exp_rel_b2, exp_root_W2)
    gflat = g.reshape(NEXP * N, D)
    outBC = _pass_bc_kernel()(gflat, src, dstB, isel[:, 0], isel[:, 1],
                              zchunk)
    y = _combine_call(ypart, outBC[0], outBC[1], wsel)
    return y
